# Initial kernel scaffold; baseline (speedup 1.0000x reference)
import jax
import jax.numpy as jnp
from jax.experimental import pallas as pl


def kernel(x, edge_index, W1, b1, W2, b2, W3, b3):
    raise NotImplementedError("write your pallas kernel here")


# speedup vs baseline: 10.8049x; 10.8049x over previous
import functools

import jax
import jax.numpy as jnp
from jax import lax
from jax.experimental import pallas as pl
from jax.experimental.pallas import tpu as pltpu
from jax.experimental.pallas import tpu_sc as plsc

N = 10000
NP = 10240
C = 128
E = 320000
NC = 2
NS = 16
L = 16
NW = NC * NS
CH = 128
K = 79
EPT = K * CH
EPAD = NW * EPT
Z = N
RPT = NP // NS
DW = 128

_mesh = plsc.VectorSubcoreMesh(
    core_axis_name="c", subcore_axis_name="s", num_cores=NC, num_subcores=NS
)


def _zero_vmem(ref, rows, width):
    zeros = jnp.zeros((L,), jnp.float32)
    groups = width // L

    def body(i, _):
        r = i // groups
        g = i % groups
        ref[r, pl.ds(g * L, L)] = zeros
        return ()

    lax.fori_loop(0, rows * groups, body, (), unroll=True)


def _make_sc_deg(dw):
    @functools.partial(
        pl.kernel,
        out_type=jax.ShapeDtypeStruct((NC * NP, dw), jnp.float32),
        mesh=_mesh,
        scratch_types=[
            pltpu.MemorySpace.VMEM((K, CH), jnp.int32),
            pltpu.MemorySpace.VMEM((CH, dw), jnp.float32),
            pltpu.MemorySpace.VMEM((CH, dw), jnp.float32),
            pltpu.MemorySpace.VMEM_SHARED((NP, dw), jnp.float32),
        ],
    )
    def sc_deg(dst_hbm, out_hbm, dst_v, ones_v, zeros_v, acc_sh):
        cid = lax.axis_index("c")
        sid = lax.axis_index("s")
        wid = cid * NS + sid

        _zero_vmem(zeros_v, CH, dw)
        ones = jnp.ones((L,), jnp.float32)
        groups = dw // L
        def fill_ones(i, _):
            ones_v[i // groups, pl.ds((i % groups) * L, L)] = ones
            return ()
        lax.fori_loop(0, CH * groups, fill_ones, (), unroll=True)

        for k in range(RPT // CH):
            pltpu.sync_copy(zeros_v, acc_sh.at[pl.ds(sid * RPT + k * CH, CH)])
        pltpu.sync_copy(dst_hbm.at[wid], dst_v)
        plsc.subcore_barrier()

        def body(ci, _):
            pltpu.sync_copy(ones_v, acc_sh.at[dst_v.at[ci]], add=True)
            return ()
        lax.fori_loop(0, K, body, ())

        plsc.subcore_barrier()
        pltpu.sync_copy(
            acc_sh.at[pl.ds(sid * RPT, RPT)],
            out_hbm.at[pl.ds(cid * NP + sid * RPT, RPT)],
        )

    return sc_deg


_sc_deg = _make_sc_deg(DW)


@functools.partial(
    pl.kernel,
    out_type=jax.ShapeDtypeStruct((NC * NP, C), jnp.float32),
    mesh=_mesh,
    scratch_types=[
        pltpu.MemorySpace.VMEM((K, CH), jnp.int32),
        pltpu.MemorySpace.VMEM((K, CH), jnp.int32),
        pltpu.MemorySpace.VMEM((CH, C), jnp.float32),
        pltpu.MemorySpace.VMEM_SHARED((NP, C), jnp.float32),
        pltpu.SemaphoreType.DMA,
    ],
)
def _sc_agg(hs_hbm, src_hbm, dst_hbm, out_hbm, src_v, dst_v, rows_v, acc_sh, sem):
    cid = lax.axis_index("c")
    sid = lax.axis_index("s")
    wid = cid * NS + sid

    _zero_vmem(rows_v, CH, C)
    for k in range(RPT // CH):
        pltpu.sync_copy(rows_v, acc_sh.at[pl.ds(sid * RPT + k * CH, CH)])
    pltpu.sync_copy(src_hbm.at[wid], src_v)
    pltpu.sync_copy(dst_hbm.at[wid], dst_v)
    plsc.subcore_barrier()

    def body(ci, _):
        pltpu.async_copy(hs_hbm.at[src_v.at[ci]], rows_v, sem).wait()
        pltpu.sync_copy(rows_v, acc_sh.at[dst_v.at[ci]], add=True)
        return ()
    lax.fori_loop(0, K, body, ())

    plsc.subcore_barrier()
    pltpu.sync_copy(
        acc_sh.at[pl.ds(sid * RPT, RPT)],
        out_hbm.at[pl.ds(cid * NP + sid * RPT, RPT)],
    )


_RB = 1024
_GRID = NP // _RB


def _dinv_body(deg_ref, o_ref):
    d = deg_ref[0 * NP:0 * NP + NP, 0:1] + deg_ref[1 * NP:1 * NP + NP, 0:1] + 1.0
    r = lax.rsqrt(d)
    row = lax.broadcasted_iota(jnp.int32, (NP, 1), 0)
    o_ref[...] = jnp.where(row < N, r, 0.0)


def _tc_dinv(deg):
    return pl.pallas_call(
        _dinv_body,
        out_shape=jax.ShapeDtypeStruct((NP, 1), jnp.float32),
    )(deg)


def _first_body(x_ref, w_ref, dinv_ref, o_ref):
    h = jnp.dot(x_ref[...], w_ref[...], preferred_element_type=jnp.float32)
    o_ref[...] = h * dinv_ref[...]


def _tc_first(x, W, dinv):
    return pl.pallas_call(
        _first_body,
        grid=(_GRID,),
        in_specs=[
            pl.BlockSpec((_RB, C), lambda i: (i, 0)),
            pl.BlockSpec((C, C), lambda i: (0, 0)),
            pl.BlockSpec((_RB, 1), lambda i: (i, 0)),
        ],
        out_specs=pl.BlockSpec((_RB, C), lambda i: (i, 0)),
        out_shape=jax.ShapeDtypeStruct((NP, C), jnp.float32),
    )(x, W, dinv)


def _mid_body(s0_ref, s1_ref, hsp_ref, dinv_ref, b_ref, w_ref, o_ref):
    t = (s0_ref[...] + s1_ref[...] + hsp_ref[...]) * dinv_ref[...] + b_ref[...]
    a = jnp.maximum(t, 0.0)
    h = jnp.dot(a, w_ref[...], preferred_element_type=jnp.float32)
    o_ref[...] = h * dinv_ref[...]


_NPB = NP // _RB


def _tc_mid(S, hsp, dinv, b, W):
    return pl.pallas_call(
        _mid_body,
        grid=(_GRID,),
        in_specs=[
            pl.BlockSpec((_RB, C), lambda i: (i, 0)),
            pl.BlockSpec((_RB, C), lambda i: (i + _NPB, 0)),
            pl.BlockSpec((_RB, C), lambda i: (i, 0)),
            pl.BlockSpec((_RB, 1), lambda i: (i, 0)),
            pl.BlockSpec((1, C), lambda i: (0, 0)),
            pl.BlockSpec((C, C), lambda i: (0, 0)),
        ],
        out_specs=pl.BlockSpec((_RB, C), lambda i: (i, 0)),
        out_shape=jax.ShapeDtypeStruct((NP, C), jnp.float32),
    )(S, S, hsp, dinv, b, W)


def _last_body(s0_ref, s1_ref, hsp_ref, dinv_ref, b_ref, o_ref):
    t = (s0_ref[...] + s1_ref[...] + hsp_ref[...]) * dinv_ref[...] + b_ref[...]
    m = jnp.max(t, axis=1, keepdims=True)
    e = jnp.exp(t - m)
    lse = jnp.log(jnp.sum(e, axis=1, keepdims=True))
    o_ref[...] = t - m - lse


def _tc_last(S, hsp, dinv, b):
    return pl.pallas_call(
        _last_body,
        grid=(_GRID,),
        in_specs=[
            pl.BlockSpec((_RB, C), lambda i: (i, 0)),
            pl.BlockSpec((_RB, C), lambda i: (i + _NPB, 0)),
            pl.BlockSpec((_RB, C), lambda i: (i, 0)),
            pl.BlockSpec((_RB, 1), lambda i: (i, 0)),
            pl.BlockSpec((1, C), lambda i: (0, 0)),
        ],
        out_specs=pl.BlockSpec((_RB, C), lambda i: (i, 0)),
        out_shape=jax.ShapeDtypeStruct((NP, C), jnp.float32),
    )(S, S, hsp, dinv, b)


def kernel(x, edge_index, W1, b1, W2, b2, W3, b3):
    src = edge_index[0].astype(jnp.int32)
    dst = edge_index[1].astype(jnp.int32)
    pad = jnp.full((EPAD - E,), Z, jnp.int32)
    srcp = jnp.concatenate([src, pad]).reshape(NW, K, CH)
    dstp = jnp.concatenate([dst, pad]).reshape(NW, K, CH)
    x_pad = jnp.pad(x, ((0, NP - N), (0, 0)))

    deg = _sc_deg(dstp)
    dinv = _tc_dinv(deg)

    b1r = b1.reshape(1, C)
    b2r = b2.reshape(1, C)
    b3r = b3.reshape(1, C)

    hs1 = _tc_first(x_pad, W1, dinv)
    S1 = _sc_agg(hs1, srcp, dstp)
    hs2 = _tc_mid(S1, hs1, dinv, b1r, W2)
    S2 = _sc_agg(hs2, srcp, dstp)
    hs3 = _tc_mid(S2, hs2, dinv, b2r, W3)
    S3 = _sc_agg(hs3, srcp, dstp)
    out = _tc_last(S3, hs3, dinv, b3r)
    return out[:N]
